# baseline (device time: 29288 ns/iter reference)
import jax
import jax.numpy as jnp
from jax import lax
from jax.experimental import pallas as pl
from jax.experimental.pallas import tpu as pltpu

N_DEV = 4
B, Sq, Skv, Hq, Dh = 2, 256, 256, 16, 64
H_LOC = Hq // N_DEV
D_MODEL = 512
BLK = 64
M = B * Sq
MC = M // N_DEV
SQC = MC


def _chunk_partial(tgt, my, x_ref, wq_ref, k_ref, v_ref, wo_ref):
    b = lax.div(tgt, jnp.int32(B))
    sq0 = lax.rem(tgt, jnp.int32(2)) * SQC

    xq = x_ref[pl.ds(tgt * MC, MC), :]
    q = jnp.dot(xq, wq_ref[...], preferred_element_type=jnp.float32)

    rows = jax.lax.broadcasted_iota(jnp.int32, (MC, Skv), 0)
    cols = jax.lax.broadcasted_iota(jnp.int32, (MC, Skv), 1)
    mask = (cols // BLK) <= ((sq0 + rows) // BLK)

    acc = jnp.zeros((MC, D_MODEL), dtype=jnp.float32)
    for h in range(H_LOC):
        h_abs = my * H_LOC + h
        qh = q[:, h * Dh:(h + 1) * Dh]
        kh = k_ref[b, :, h_abs, :]
        vh = v_ref[b, :, h_abs, :]
        s = lax.dot_general(
            qh, kh, (((1,), (1,)), ((), ())),
            preferred_element_type=jnp.float32,
        ) * 0.125
        s = jnp.where(mask, s, -1e9)
        m = jnp.max(s, axis=-1, keepdims=True)
        e = jnp.exp(s - m)
        w = e / jnp.sum(e, axis=-1, keepdims=True)
        ctxh = jnp.dot(w, vh, preferred_element_type=jnp.float32)
        acc = acc + jnp.dot(
            ctxh, wo_ref[pl.ds(h * Dh, Dh), :],
            preferred_element_type=jnp.float32,
        )
    return acc


def kernel(x, Wq, K_ext, V_ext, Wo):
    x2 = x.reshape(M, D_MODEL)

    def body(x_ref, wq_ref, k_ref, v_ref, wo_ref, out_ref,
             send_src, rs_buf, ag_src, ag_buf,
             rs_send, rs_recv, ag_send, ag_recv):
        my = lax.axis_index("i")

        barrier_sem = pltpu.get_barrier_semaphore()
        for d in range(1, N_DEV):
            tgt = lax.rem(my + d, N_DEV)
            pl.semaphore_signal(
                barrier_sem, inc=1,
                device_id=(tgt,), device_id_type=pl.DeviceIdType.MESH,
            )
        pl.semaphore_wait(barrier_sem, N_DEV - 1)

        rs_rdmas = []
        for d in range(1, N_DEV):
            tgt = lax.rem(my + d, N_DEV)
            send_src[d - 1] = _chunk_partial(
                tgt, my, x_ref, wq_ref, k_ref, v_ref, wo_ref)
            rdma = pltpu.make_async_remote_copy(
                src_ref=send_src.at[d - 1],
                dst_ref=rs_buf.at[d - 1],
                send_sem=rs_send.at[d - 1],
                recv_sem=rs_recv.at[d - 1],
                device_id=(tgt,),
                device_id_type=pl.DeviceIdType.MESH,
            )
            rdma.start()
            rs_rdmas.append(rdma)

        chunk = _chunk_partial(my, my, x_ref, wq_ref, k_ref, v_ref, wo_ref)

        for rdma in rs_rdmas:
            rdma.wait_recv()

        for kk in range(N_DEV - 1):
            chunk = chunk + rs_buf[kk]
        ag_src[...] = chunk
        out_ref[pl.ds(my * MC, MC), :] = chunk

        ag_rdmas = []
        for d in range(1, N_DEV):
            tgt = lax.rem(my + d, N_DEV)
            rdma = pltpu.make_async_remote_copy(
                src_ref=ag_src,
                dst_ref=ag_buf.at[d - 1],
                send_sem=ag_send.at[d - 1],
                recv_sem=ag_recv.at[d - 1],
                device_id=(tgt,),
                device_id_type=pl.DeviceIdType.MESH,
            )
            rdma.start()
            ag_rdmas.append(rdma)

        for d in range(1, N_DEV):
            ag_rdmas[d - 1].wait_recv()
            src_dev = lax.rem(my + N_DEV - d, N_DEV)
            out_ref[pl.ds(src_dev * MC, MC), :] = ag_buf[d - 1]

        for rdma in rs_rdmas:
            rdma.wait_send()
        for rdma in ag_rdmas:
            rdma.wait_send()

    out = pl.pallas_call(
        body,
        out_shape=jax.ShapeDtypeStruct((M, D_MODEL), jnp.float32),
        in_specs=[pl.BlockSpec(memory_space=pltpu.VMEM)] * 5,
        out_specs=pl.BlockSpec(memory_space=pltpu.VMEM),
        scratch_shapes=[
            pltpu.VMEM((N_DEV - 1, MC, D_MODEL), jnp.float32),
            pltpu.VMEM((N_DEV - 1, MC, D_MODEL), jnp.float32),
            pltpu.VMEM((MC, D_MODEL), jnp.float32),
            pltpu.VMEM((N_DEV - 1, MC, D_MODEL), jnp.float32),
            pltpu.SemaphoreType.DMA((N_DEV - 1,)),
            pltpu.SemaphoreType.DMA((N_DEV - 1,)),
            pltpu.SemaphoreType.DMA((N_DEV - 1,)),
            pltpu.SemaphoreType.DMA((N_DEV - 1,)),
        ],
        compiler_params=pltpu.CompilerParams(collective_id=0),
    )(x2, Wq, K_ext, V_ext, Wo)
    return out.reshape(B, Sq, D_MODEL)


# device time: 23841 ns/iter; 1.2285x vs baseline; 1.2285x over previous
import jax
import jax.numpy as jnp
from jax import lax
from jax.experimental import pallas as pl
from jax.experimental.pallas import tpu as pltpu

N_DEV = 4
B, Sq, Skv, Hq, Dh = 2, 256, 256, 16, 64
H_LOC = Hq // N_DEV
D_MODEL = 512
BLK = 64
N_SUB = 2
D_ORDER = (2, 1, 3)


def _allreduce(partial):
    m, n = partial.shape
    mc = m // N_DEV
    ms = mc // N_SUB

    def body(p_ref, out_ref, rs_buf, ag_src, ag_buf,
             rs_send, rs_recv, ag_send, ag_recv):
        my = lax.axis_index("i")

        barrier_sem = pltpu.get_barrier_semaphore()
        for d in range(1, N_DEV):
            tgt = lax.rem(my + d, N_DEV)
            pl.semaphore_signal(
                barrier_sem, inc=1,
                device_id=(tgt,), device_id_type=pl.DeviceIdType.MESH,
            )
        pl.semaphore_wait(barrier_sem, N_DEV - 1)

        rs_rdmas = {}
        for s in range(N_SUB):
            for d in D_ORDER:
                tgt = lax.rem(my + d, N_DEV)
                rdma = pltpu.make_async_remote_copy(
                    src_ref=p_ref.at[pl.ds(tgt * mc + s * ms, ms), :],
                    dst_ref=rs_buf.at[d - 1, s],
                    send_sem=rs_send.at[d - 1, s],
                    recv_sem=rs_recv.at[d - 1, s],
                    device_id=(tgt,),
                    device_id_type=pl.DeviceIdType.MESH,
                )
                rdma.start()
                rs_rdmas[(d, s)] = rdma

        ag_rdmas = {}
        for s in range(N_SUB):
            for d in D_ORDER:
                rs_rdmas[(d, s)].wait_recv()
            chunk = p_ref[pl.ds(my * mc + s * ms, ms), :]
            for k in range(N_DEV - 1):
                chunk = chunk + rs_buf[k, s]
            ag_src[s] = chunk
            out_ref[pl.ds(my * mc + s * ms, ms), :] = chunk
            for d in D_ORDER:
                tgt = lax.rem(my + d, N_DEV)
                rdma = pltpu.make_async_remote_copy(
                    src_ref=ag_src.at[s],
                    dst_ref=ag_buf.at[d - 1, s],
                    send_sem=ag_send.at[d - 1, s],
                    recv_sem=ag_recv.at[d - 1, s],
                    device_id=(tgt,),
                    device_id_type=pl.DeviceIdType.MESH,
                )
                rdma.start()
                ag_rdmas[(d, s)] = rdma

        for s in range(N_SUB):
            for d in range(1, N_DEV):
                ag_rdmas[(d, s)].wait_recv()
                src_dev = lax.rem(my + N_DEV - d, N_DEV)
                out_ref[pl.ds(src_dev * mc + s * ms, ms), :] = ag_buf[d - 1, s]

        for rdma in rs_rdmas.values():
            rdma.wait_send()
        for rdma in ag_rdmas.values():
            rdma.wait_send()

    return pl.pallas_call(
        body,
        out_shape=jax.ShapeDtypeStruct((m, n), jnp.float32),
        in_specs=[pl.BlockSpec(memory_space=pltpu.VMEM)],
        out_specs=pl.BlockSpec(memory_space=pltpu.VMEM),
        scratch_shapes=[
            pltpu.VMEM((N_DEV - 1, N_SUB, ms, n), jnp.float32),
            pltpu.VMEM((N_SUB, ms, n), jnp.float32),
            pltpu.VMEM((N_DEV - 1, N_SUB, ms, n), jnp.float32),
            pltpu.SemaphoreType.DMA((N_DEV - 1, N_SUB)),
            pltpu.SemaphoreType.DMA((N_DEV - 1, N_SUB)),
            pltpu.SemaphoreType.DMA((N_DEV - 1, N_SUB)),
            pltpu.SemaphoreType.DMA((N_DEV - 1, N_SUB)),
        ],
        compiler_params=pltpu.CompilerParams(collective_id=0),
    )(partial)


def kernel(x, Wq, K_ext, V_ext, Wo):
    i = lax.axis_index("i")

    Q = (x.reshape(B * Sq, D_MODEL) @ Wq).reshape(B, Sq, H_LOC, Dh)
    K = lax.dynamic_slice_in_dim(K_ext, i * H_LOC, H_LOC, axis=2)
    V = lax.dynamic_slice_in_dim(V_ext, i * H_LOC, H_LOC, axis=2)

    scores = jnp.einsum("bihd,bjhd->bhij", Q, K) * 0.125
    qb = jnp.arange(Sq) // BLK
    kb = jnp.arange(Skv) // BLK
    mask = kb[None, :] <= qb[:, None]
    scores = jnp.where(mask[None, None], scores, -1e9)
    w = jax.nn.softmax(scores, axis=-1)
    ctx = jnp.einsum("bhij,bjhd->bihd", w, V).reshape(B * Sq, H_LOC * Dh)

    partial = ctx @ Wo
    out = _allreduce(partial)
    return out.reshape(B, Sq, D_MODEL)
